# Initial kernel scaffold; baseline (speedup 1.0000x reference)
import functools

import jax
import jax.numpy as jnp
from jax import lax
from jax.experimental import pallas as pl
from jax.experimental.pallas import tpu as pltpu
from jax.experimental.pallas import tpu_sc as plsc

N_NODES = 10000
N_EDGES = 320000
GATE = 128
REG = 16
INTER = 72
PAD = 80
NEW = 128

E_BLK = 80
N_WORKERS = 32
EDGES_PER_W = N_EDGES // N_WORKERS
BLKS_PER_W = EDGES_PER_W // E_BLK
ROWS_PER_TILE = N_NODES // 16
ZROWS = 125

NODE_BLK = 1000
EDGE_BLK = 3200


def _node_proj_body(x_ref, w_ref, o_ref):
    acc = jnp.dot(x_ref[...], w_ref[...], preferred_element_type=jnp.float32)
    col = lax.broadcasted_iota(jnp.int32, acc.shape, 1)
    o_ref[...] = acc + jnp.where(col == INTER, 1.0, 0.0)


def _edge_proj_body(e_ref, w_ref, o_ref):
    o_ref[...] = jnp.dot(e_ref[...], w_ref[...], preferred_element_type=jnp.float32)


def _final_body(p_ref, x_ref, w2x_ref, w2m_ref, b2_ref, o_ref):
    s = p_ref[0] + p_ref[1]
    deg = s[:, INTER:INTER + 1]
    mean = s / jnp.maximum(deg, 1.0)
    h = (jnp.dot(x_ref[...], w2x_ref[...], preferred_element_type=jnp.float32)
         + jnp.dot(mean, w2m_ref[...], preferred_element_type=jnp.float32)
         + b2_ref[...])
    o_ref[...] = jnp.where(h >= 0, h, 0.01 * h)


def _sc_segment_sum(a_pad, b_pad, src, dst):
    mesh = plsc.VectorSubcoreMesh(core_axis_name="c", subcore_axis_name="s")

    @functools.partial(
        pl.kernel,
        mesh=mesh,
        out_type=jax.ShapeDtypeStruct((2, N_NODES, PAD), jnp.float32),
        scratch_types=[
            pltpu.VMEM((E_BLK,), jnp.int32),
            pltpu.VMEM((E_BLK,), jnp.int32),
            pltpu.VMEM((E_BLK, PAD), jnp.float32),
            pltpu.VMEM((E_BLK, PAD), jnp.float32),
            pltpu.VMEM((ZROWS, PAD), jnp.float32),
            pltpu.VMEM_SHARED((N_NODES, PAD), jnp.float32),
            pltpu.SemaphoreType.DMA,
        ],
    )
    def sc_kernel(a_hbm, b_hbm, src_hbm, dst_hbm, out_hbm,
                  sidx, didx, arows, brows, zbuf, acc, sem):
        c = lax.axis_index("c")
        s = lax.axis_index("s")
        wid = s * 2 + c

        zero = jnp.zeros((16,), jnp.float32)

        def zrow(r, carry):
            for j in range(PAD // 16):
                zbuf[r, pl.ds(j * 16, 16)] = zero
            return carry

        lax.fori_loop(0, ZROWS, zrow, 0)
        for k in range(ROWS_PER_TILE // ZROWS):
            pltpu.sync_copy(
                zbuf, acc.at[pl.ds(s * ROWS_PER_TILE + k * ZROWS, ZROWS)])
        plsc.subcore_barrier()

        base0 = wid * EDGES_PER_W

        def blk(i, carry):
            base = base0 + i * E_BLK
            pltpu.sync_copy(src_hbm.at[pl.ds(base, E_BLK)], sidx)
            pltpu.sync_copy(dst_hbm.at[pl.ds(base, E_BLK)], didx)
            pltpu.async_copy(a_hbm.at[sidx], arows, sem).wait()
            pltpu.sync_copy(b_hbm.at[pl.ds(base, E_BLK)], brows)

            def row(r, rcarry):
                for j in range(PAD // 16):
                    v = (arows[r, pl.ds(j * 16, 16)]
                         + brows[r, pl.ds(j * 16, 16)])
                    brows[r, pl.ds(j * 16, 16)] = jnp.where(v >= 0, v, 0.01 * v)
                return rcarry

            lax.fori_loop(0, E_BLK, row, 0)
            pltpu.sync_copy(brows, acc.at[didx], add=True)
            return carry

        lax.fori_loop(0, BLKS_PER_W, blk, 0)
        plsc.subcore_barrier()

        pltpu.sync_copy(acc.at[pl.ds(s * ROWS_PER_TILE, ROWS_PER_TILE)],
                        out_hbm.at[c, pl.ds(s * ROWS_PER_TILE, ROWS_PER_TILE)])

    return sc_kernel(a_pad, b_pad, src, dst)


def kernel(input_gate_embedding, edge_index, edge_reg_embedding, W1, W2, b2):
    x = input_gate_embedding
    src = edge_index[0].astype(jnp.int32)
    dst = edge_index[1].astype(jnp.int32)

    w1xT = jnp.pad(W1[:, :GATE].T, ((0, 0), (0, PAD - INTER)))
    w1eT = jnp.pad(W1[:, GATE:].T, ((0, 0), (0, PAD - INTER)))
    w2xT = W2[:, :GATE].T
    w2mT = jnp.pad(W2[:, GATE:].T, ((0, PAD - INTER), (0, 0)))
    b2r = b2.reshape(1, NEW)

    a_pad = pl.pallas_call(
        _node_proj_body,
        grid=(N_NODES // NODE_BLK,),
        in_specs=[
            pl.BlockSpec((NODE_BLK, GATE), lambda i: (i, 0)),
            pl.BlockSpec((GATE, PAD), lambda i: (0, 0)),
        ],
        out_specs=pl.BlockSpec((NODE_BLK, PAD), lambda i: (i, 0)),
        out_shape=jax.ShapeDtypeStruct((N_NODES, PAD), jnp.float32),
    )(x, w1xT)

    b_pad = pl.pallas_call(
        _edge_proj_body,
        grid=(N_EDGES // EDGE_BLK,),
        in_specs=[
            pl.BlockSpec((EDGE_BLK, REG), lambda i: (i, 0)),
            pl.BlockSpec((REG, PAD), lambda i: (0, 0)),
        ],
        out_specs=pl.BlockSpec((EDGE_BLK, PAD), lambda i: (i, 0)),
        out_shape=jax.ShapeDtypeStruct((N_EDGES, PAD), jnp.float32),
    )(edge_reg_embedding, w1eT)

    partials = _sc_segment_sum(a_pad, b_pad, src, dst)

    out = pl.pallas_call(
        _final_body,
        grid=(N_NODES // NODE_BLK,),
        in_specs=[
            pl.BlockSpec((2, NODE_BLK, PAD), lambda i: (0, i, 0)),
            pl.BlockSpec((NODE_BLK, GATE), lambda i: (i, 0)),
            pl.BlockSpec((GATE, NEW), lambda i: (0, 0)),
            pl.BlockSpec((PAD, NEW), lambda i: (0, 0)),
            pl.BlockSpec((1, NEW), lambda i: (0, 0)),
        ],
        out_specs=pl.BlockSpec((NODE_BLK, NEW), lambda i: (i, 0)),
        out_shape=jax.ShapeDtypeStruct((N_NODES, NEW), jnp.float32),
    )(partials, x, w2xT, w2mT, b2r)

    return out


# speedup vs baseline: 5.7085x; 5.7085x over previous
import functools

import jax
import jax.numpy as jnp
from jax import lax
from jax.experimental import pallas as pl
from jax.experimental.pallas import tpu as pltpu
from jax.experimental.pallas import tpu_sc as plsc

N_NODES = 10000
N_EDGES = 320000
GATE = 128
REG = 16
INTER = 72
PAD = 80
BW = 128
NEW = 128

E_BLK = 80
N_WORKERS = 32
EDGES_PER_W = N_EDGES // N_WORKERS
NBLK = EDGES_PER_W // E_BLK
ACC_ROWS = 10240
ROWS_PER_TILE = ACC_ROWS // 16
ZROWS = 128

NODE_BLK = 1000
EDGE_BLK = 3200


def _node_proj_body(x_ref, w_ref, o_ref):
    acc = jnp.dot(x_ref[...], w_ref[...], preferred_element_type=jnp.float32)
    col = lax.broadcasted_iota(jnp.int32, acc.shape, 1)
    o_ref[...] = acc + jnp.where(col == INTER, 1.0, 0.0)


def _edge_proj_body(e_ref, w_ref, o_ref):
    o_ref[...] = jnp.dot(e_ref[...], w_ref[...], preferred_element_type=jnp.float32)


def _final_body(p_ref, x_ref, w2x_ref, w2m_ref, b2_ref, o_ref):
    s = p_ref[0] + p_ref[1]
    deg = s[:, INTER:INTER + 1]
    mean = s / jnp.maximum(deg, 1.0)
    h = (jnp.dot(x_ref[...], w2x_ref[...], preferred_element_type=jnp.float32)
         + jnp.dot(mean, w2m_ref[...], preferred_element_type=jnp.float32)
         + b2_ref[...])
    o_ref[...] = jnp.where(h >= 0, h, 0.01 * h)


def _sc_segment_sum(a_pad, b_pad, src2d, dst2d):
    mesh = plsc.VectorSubcoreMesh(core_axis_name="c", subcore_axis_name="s")

    @functools.partial(
        pl.kernel,
        mesh=mesh,
        out_type=jax.ShapeDtypeStruct((2, ACC_ROWS, PAD), jnp.float32),
        scratch_types=[
            pltpu.VMEM((NBLK, E_BLK), jnp.int32),
            pltpu.VMEM((NBLK, E_BLK), jnp.int32),
            pltpu.VMEM((E_BLK, PAD), jnp.float32),
            pltpu.VMEM((E_BLK, PAD), jnp.float32),
            pltpu.VMEM((E_BLK, BW), jnp.float32),
            pltpu.VMEM((E_BLK, BW), jnp.float32),
            pltpu.VMEM((E_BLK, PAD), jnp.float32),
            pltpu.VMEM((ZROWS, PAD), jnp.float32),
            pltpu.VMEM_SHARED((ACC_ROWS, PAD), jnp.float32),
            pltpu.SemaphoreType.DMA,
            pltpu.SemaphoreType.DMA,
            pltpu.SemaphoreType.DMA,
            pltpu.SemaphoreType.DMA,
        ],
        compiler_params=pltpu.CompilerParams(use_tc_tiling_on_sc=False),
    )
    def sc_kernel(a_hbm, b_hbm, src_hbm, dst_hbm, out_hbm,
                  sidx, didx, ar0, ar1, br0, br1, tbuf, zbuf, acc,
                  gs0, gs1, bs0, bs1):
        c = lax.axis_index("c")
        s = lax.axis_index("s")
        wid = s * 2 + c

        zero = jnp.zeros((16,), jnp.float32)

        @plsc.parallel_loop(0, ZROWS)
        def _zrow(r):
            for j in range(PAD // 16):
                zbuf[r, pl.ds(j * 16, 16)] = zero

        for k in range(ROWS_PER_TILE // ZROWS):
            pltpu.sync_copy(
                zbuf, acc.at[pl.ds(s * ROWS_PER_TILE + k * ZROWS, ZROWS)])
        plsc.subcore_barrier()

        base_e = wid * EDGES_PER_W

        pltpu.sync_copy(src_hbm.at[pl.ds(wid * NBLK, NBLK)], sidx)
        pltpu.sync_copy(dst_hbm.at[pl.ds(wid * NBLK, NBLK)], didx)

        def start(b, ar, br, gsem, bsem):
            pltpu.async_copy(a_hbm.at[sidx.at[b]], ar, gsem)
            pltpu.async_copy(b_hbm.at[pl.ds(base_e + b * E_BLK, E_BLK)], br, bsem)

        def process(b, ar, br, gsem, bsem):
            pltpu.make_async_copy(a_hbm.at[sidx.at[b]], ar, gsem).wait()
            pltpu.make_async_copy(
                b_hbm.at[pl.ds(base_e + b * E_BLK, E_BLK)], br, bsem).wait()

            @plsc.parallel_loop(0, E_BLK, unroll=2)
            def _row(r):
                for j in range(PAD // 16):
                    v = ar[r, pl.ds(j * 16, 16)] + br[r, pl.ds(j * 16, 16)]
                    tbuf[r, pl.ds(j * 16, 16)] = jnp.where(v >= 0, v, 0.01 * v)

            pltpu.sync_copy(tbuf, acc.at[didx.at[b]], add=True)

            @pl.when(b + 2 < NBLK)
            def _():
                start(b + 2, ar, br, gsem, bsem)

        start(0, ar0, br0, gs0, bs0)
        start(1, ar1, br1, gs1, bs1)

        def pair(j, carry):
            b0 = 2 * j
            process(b0, ar0, br0, gs0, bs0)

            @pl.when(b0 + 1 < NBLK)
            def _():
                process(b0 + 1, ar1, br1, gs1, bs1)

            return carry

        lax.fori_loop(0, (NBLK + 1) // 2, pair, 0)
        plsc.subcore_barrier()

        pltpu.sync_copy(acc.at[pl.ds(s * ROWS_PER_TILE, ROWS_PER_TILE)],
                        out_hbm.at[c, pl.ds(s * ROWS_PER_TILE, ROWS_PER_TILE)])

    return sc_kernel(a_pad, b_pad, src2d, dst2d)


def kernel(input_gate_embedding, edge_index, edge_reg_embedding, W1, W2, b2):
    x = input_gate_embedding
    src2d = edge_index[0].astype(jnp.int32).reshape(N_EDGES // E_BLK, E_BLK)
    dst2d = edge_index[1].astype(jnp.int32).reshape(N_EDGES // E_BLK, E_BLK)

    w1xT = jnp.pad(W1[:, :GATE].T, ((0, 0), (0, PAD - INTER)))
    w1eT = jnp.pad(W1[:, GATE:].T, ((0, 0), (0, BW - INTER)))
    w2xT = W2[:, :GATE].T
    w2mT = jnp.pad(W2[:, GATE:].T, ((0, PAD - INTER), (0, 0)))
    b2r = b2.reshape(1, NEW)

    a_pad = pl.pallas_call(
        _node_proj_body,
        grid=(N_NODES // NODE_BLK,),
        in_specs=[
            pl.BlockSpec((NODE_BLK, GATE), lambda i: (i, 0)),
            pl.BlockSpec((GATE, PAD), lambda i: (0, 0)),
        ],
        out_specs=pl.BlockSpec((NODE_BLK, PAD), lambda i: (i, 0)),
        out_shape=jax.ShapeDtypeStruct((N_NODES, PAD), jnp.float32),
    )(x, w1xT)

    b_pad = pl.pallas_call(
        _edge_proj_body,
        grid=(N_EDGES // EDGE_BLK,),
        in_specs=[
            pl.BlockSpec((EDGE_BLK, REG), lambda i: (i, 0)),
            pl.BlockSpec((REG, BW), lambda i: (0, 0)),
        ],
        out_specs=pl.BlockSpec((EDGE_BLK, BW), lambda i: (i, 0)),
        out_shape=jax.ShapeDtypeStruct((N_EDGES, BW), jnp.float32),
    )(edge_reg_embedding, w1eT)

    partials = _sc_segment_sum(a_pad, b_pad, src2d, dst2d)

    out = pl.pallas_call(
        _final_body,
        grid=(N_NODES // NODE_BLK,),
        in_specs=[
            pl.BlockSpec((2, NODE_BLK, PAD), lambda i: (0, i, 0)),
            pl.BlockSpec((NODE_BLK, GATE), lambda i: (i, 0)),
            pl.BlockSpec((GATE, NEW), lambda i: (0, 0)),
            pl.BlockSpec((PAD, NEW), lambda i: (0, 0)),
            pl.BlockSpec((1, NEW), lambda i: (0, 0)),
        ],
        out_specs=pl.BlockSpec((NODE_BLK, NEW), lambda i: (i, 0)),
        out_shape=jax.ShapeDtypeStruct((N_NODES, NEW), jnp.float32),
    )(partials, x, w2xT, w2mT, b2r)

    return out
